# baseline (speedup 1.0000x reference)
import functools
import jax
import jax.numpy as jnp
from jax import lax
from jax.experimental import pallas as pl
from jax.experimental.pallas import tpu as pltpu
from jax.experimental.pallas import tpu_sc as plsc

_B, _P, _D = 64, 576, 768
_NC, _NS = 2, 16
_NW = _NC * _NS
_PW = _P // _NW
_CH = _PW * _D
_NV = _CH // 16

_mesh = plsc.VectorSubcoreMesh(core_axis_name="c", subcore_axis_name="s")


@functools.partial(
    pl.kernel,
    mesh=_mesh,
    out_type=jax.ShapeDtypeStruct((_B, _P * _D), jnp.float32),
    scratch_types=[
        pltpu.VMEM((_CH,), jnp.float32),
        pltpu.VMEM((_CH,), jnp.float32),
        pltpu.VMEM((_CH,), jnp.float32),
        pltpu.VMEM((_CH,), jnp.float32),
        pltpu.VMEM((_CH,), jnp.float32),
        pltpu.SemaphoreType.DMA,
        pltpu.SemaphoreType.DMA,
        pltpu.SemaphoreType.DMA,
        pltpu.SemaphoreType.DMA,
    ],
)
def _sc_add(enc_hbm, pos_hbm, out_hbm, pos_v, ib0, ib1, ob0, ob1, si0, si1, so0, so1):
    wid = lax.axis_index("s") * _NC + lax.axis_index("c")
    base = wid * _CH
    sl = pl.ds(base, _CH)
    pltpu.sync_copy(pos_hbm.at[sl], pos_v)

    pltpu.async_copy(enc_hbm.at[0, sl], ib0, si0)
    pltpu.async_copy(enc_hbm.at[1, sl], ib1, si1)

    def halfstep(b, ib, ob, si, so):
        pltpu.make_async_copy(enc_hbm.at[b, sl], ib, si).wait()

        @pl.when(b >= 2)
        def _():
            pltpu.make_async_copy(ob, out_hbm.at[b, sl], so).wait()

        def add_body(i, c):
            s = pl.ds(i * 16, 16)
            ob[s] = ib[s] + pos_v[s]
            return c

        lax.fori_loop(0, _NV, add_body, 0)

        pltpu.async_copy(ob, out_hbm.at[b, sl], so)

        @pl.when(b + 2 < _B)
        def _():
            pltpu.async_copy(enc_hbm.at[b + 2, sl], ib, si)

    def body(i, carry):
        halfstep(2 * i, ib0, ob0, si0, so0)
        halfstep(2 * i + 1, ib1, ob1, si1, so1)
        return carry

    lax.fori_loop(0, _B // 2, body, 0)

    pltpu.make_async_copy(ob0, out_hbm.at[_B - 2, sl], so0).wait()
    pltpu.make_async_copy(ob1, out_hbm.at[_B - 1, sl], so1).wait()


def kernel(encoded_patches, pos_table):
    enc2 = encoded_patches.reshape(_B, _P * _D)
    pos1 = pos_table.reshape(_P * _D)
    out = _sc_add(enc2, pos1)
    return out.reshape(_B, _P, _D)


# speedup vs baseline: 1.4721x; 1.4721x over previous
import functools
import jax
import jax.numpy as jnp
from jax import lax
from jax.experimental import pallas as pl
from jax.experimental.pallas import tpu as pltpu
from jax.experimental.pallas import tpu_sc as plsc

_B, _P, _D = 64, 576, 768
_K = 16
_NC, _NS = 2, 16
_NW = _NC * _NS
_PW = _P // _NW
_CH = _PW * _D
_NV = _CH // 16
_UNROLL = 8

_mesh = plsc.VectorSubcoreMesh(core_axis_name="c", subcore_axis_name="s")


@functools.partial(
    pl.kernel,
    mesh=_mesh,
    out_type=jax.ShapeDtypeStruct((_K, _P * _D), jnp.float32),
    scratch_types=[
        pltpu.VMEM((_CH,), jnp.float32),
        pltpu.VMEM((_CH,), jnp.float32),
        pltpu.VMEM((_CH,), jnp.float32),
        pltpu.VMEM((_CH,), jnp.float32),
        pltpu.VMEM((_CH,), jnp.float32),
        pltpu.SemaphoreType.DMA,
        pltpu.SemaphoreType.DMA,
        pltpu.SemaphoreType.DMA,
        pltpu.SemaphoreType.DMA,
    ],
)
def _sc_add(enc_hbm, pos_hbm, out_hbm, pos_v, ib0, ib1, ob0, ob1, si0, si1, so0, so1):
    wid = lax.axis_index("s") * _NC + lax.axis_index("c")
    base = wid * _CH
    sl = pl.ds(base, _CH)
    pltpu.sync_copy(pos_hbm.at[sl], pos_v)

    pltpu.async_copy(enc_hbm.at[0, sl], ib0, si0)
    pltpu.async_copy(enc_hbm.at[1, sl], ib1, si1)

    def halfstep(b, ib, ob, si, so):
        pltpu.make_async_copy(enc_hbm.at[b, sl], ib, si).wait()

        @pl.when(b >= 2)
        def _():
            pltpu.make_async_copy(ob, out_hbm.at[b, sl], so).wait()

        def add_body(i, c):
            for k in range(_UNROLL):
                s = pl.ds(i * (16 * _UNROLL) + k * 16, 16)
                ob[s] = ib[s] + pos_v[s]
            return c

        lax.fori_loop(0, _NV // _UNROLL, add_body, 0)

        pltpu.async_copy(ob, out_hbm.at[b, sl], so)

        @pl.when(b + 2 < _K)
        def _():
            pltpu.async_copy(enc_hbm.at[b + 2, sl], ib, si)

    def body(i, carry):
        halfstep(2 * i, ib0, ob0, si0, so0)
        halfstep(2 * i + 1, ib1, ob1, si1, so1)
        return carry

    lax.fori_loop(0, _K // 2, body, 0)

    pltpu.make_async_copy(ob0, out_hbm.at[_K - 2, sl], so0).wait()
    pltpu.make_async_copy(ob1, out_hbm.at[_K - 1, sl], so1).wait()


def _tc_body(enc_ref, pos_ref, out_ref):
    out_ref[...] = enc_ref[...] + pos_ref[...]


def _tc_add(encoded_patches, pos_table):
    return pl.pallas_call(
        _tc_body,
        grid=(_B - _K,),
        in_specs=[
            pl.BlockSpec((1, _P, _D), lambda b: (b + _K, 0, 0)),
            pl.BlockSpec((_P, _D), lambda b: (0, 0)),
        ],
        out_specs=pl.BlockSpec((1, _P, _D), lambda b: (b, 0, 0)),
        out_shape=jax.ShapeDtypeStruct((_B - _K, _P, _D), jnp.float32),
    )(encoded_patches, pos_table)


def kernel(encoded_patches, pos_table):
    enc2 = encoded_patches.reshape(_B, _P * _D)
    pos1 = pos_table.reshape(_P * _D)
    out_sc = _sc_add(enc2, pos1).reshape(_K, _P, _D)
    out_tc = _tc_add(encoded_patches, pos_table)
    return jnp.concatenate([out_sc, out_tc], axis=0)


# speedup vs baseline: 1.4863x; 1.0096x over previous
import functools
import jax
import jax.numpy as jnp
from jax import lax
from jax.experimental import pallas as pl
from jax.experimental.pallas import tpu as pltpu
from jax.experimental.pallas import tpu_sc as plsc

_B, _P, _D = 64, 576, 768
_NC, _NS = 2, 16
_NW = _NC * _NS
_PW = _P // _NW
_CH = _PW * _D
_NV = _CH // 16
_UNROLL = 8

_mesh = plsc.VectorSubcoreMesh(core_axis_name="c", subcore_axis_name="s")


@functools.partial(
    pl.kernel,
    mesh=_mesh,
    out_type=jax.ShapeDtypeStruct((_B, _P * _D), jnp.float32),
    scratch_types=[
        pltpu.VMEM((_CH,), jnp.float32),
        pltpu.VMEM((_CH,), jnp.float32),
        pltpu.VMEM((_CH,), jnp.float32),
        pltpu.VMEM((_CH,), jnp.float32),
        pltpu.VMEM((_CH,), jnp.float32),
        pltpu.SemaphoreType.DMA,
        pltpu.SemaphoreType.DMA,
        pltpu.SemaphoreType.DMA,
        pltpu.SemaphoreType.DMA,
    ],
)
def _sc_add(enc_hbm, pos_hbm, out_hbm, pos_v, ib0, ib1, ob0, ob1, si0, si1, so0, so1):
    wid = lax.axis_index("s") * _NC + lax.axis_index("c")
    base = wid * _CH
    sl = pl.ds(base, _CH)
    pltpu.sync_copy(pos_hbm.at[sl], pos_v)

    pltpu.async_copy(enc_hbm.at[0, sl], ib0, si0)
    pltpu.async_copy(enc_hbm.at[1, sl], ib1, si1)

    def halfstep(b, ib, ob, si, so):
        pltpu.make_async_copy(enc_hbm.at[b, sl], ib, si).wait()

        @pl.when(b >= 2)
        def _():
            pltpu.make_async_copy(ob, out_hbm.at[b, sl], so).wait()

        def add_body(i, c):
            for k in range(_UNROLL):
                s = pl.ds(i * (16 * _UNROLL) + k * 16, 16)
                ob[s] = ib[s] + pos_v[s]
            return c

        lax.fori_loop(0, _NV // _UNROLL, add_body, 0)

        pltpu.async_copy(ob, out_hbm.at[b, sl], so)

        @pl.when(b + 2 < _B)
        def _():
            pltpu.async_copy(enc_hbm.at[b + 2, sl], ib, si)

    def body(i, carry):
        halfstep(2 * i, ib0, ob0, si0, so0)
        halfstep(2 * i + 1, ib1, ob1, si1, so1)
        return carry

    lax.fori_loop(0, _B // 2, body, 0)

    pltpu.make_async_copy(ob0, out_hbm.at[_B - 2, sl], so0).wait()
    pltpu.make_async_copy(ob1, out_hbm.at[_B - 1, sl], so1).wait()


def kernel(encoded_patches, pos_table):
    enc2 = encoded_patches.reshape(_B, _P * _D)
    pos1 = pos_table.reshape(_P * _D)
    out = _sc_add(enc2, pos1)
    return out.reshape(_B, _P, _D)
